# baseline (speedup 1.0000x reference)
import functools

import jax
import jax.numpy as jnp
from jax import lax
from jax.experimental import pallas as pl
from jax.experimental.pallas import tpu as pltpu
from jax.experimental.pallas import tpu_sc as plsc

N = 10000
E = 320000
D = 128
RT = 2 * N
RL = 10240
TRASH = N
NC = 2
NS = 16
NW = NC * NS
CH = 128
KR = 1
NCHUNK = 160
SQ = NCHUNK // KR
PW = NCHUNK * CH
EPG = NS * PW
EP = NC * EPG
DC = 2560

_HI = jax.lax.Precision.HIGHEST


def _dot(a, b):
    return jnp.dot(a, b, precision=_HI)


def _sc_mesh():
    return plsc.VectorSubcoreMesh(
        core_axis_name="c", subcore_axis_name="s",
        num_cores=NC, num_subcores=NS)


def _make_deg_kernel():
    return functools.partial(
        pl.kernel,
        out_type=jax.ShapeDtypeStruct((NW * RL,), jnp.float32),
        mesh=_sc_mesh(),
        scratch_types=[
            pltpu.VMEM((DC,), jnp.int32),
            pltpu.VMEM((RL,), jnp.float32),
        ],
        compiler_params=pltpu.CompilerParams(needs_layout_passes=False),
    )(_deg_body)


def _deg_body(dst_hbm, out, dstb, degv):
    wid = lax.axis_index("c") * NS + lax.axis_index("s")
    zeros16 = jnp.zeros((16,), jnp.float32)
    ones16 = jnp.ones((16,), jnp.float32)

    def zbody(i, _):
        degv[pl.ds(pl.multiple_of(i * 16, 16), 16)] = zeros16
        return _

    lax.fori_loop(0, RL // 16, zbody, None)

    base = pl.multiple_of(wid * PW, 8)

    def chunk(jc, _):
        pltpu.sync_copy(dst_hbm.at[pl.ds(pl.multiple_of(base + jc * DC, 8), DC)], dstb)

        def inner(t, _):
            idx = dstb[pl.ds(pl.multiple_of(t * 16, 16), 16)]
            plsc.addupdate_scatter(degv, [idx], ones16)
            return _

        lax.fori_loop(0, DC // 16, inner, None)
        return _

    lax.fori_loop(0, PW // DC, chunk, None)
    pltpu.sync_copy(degv, out.at[pl.ds(pl.multiple_of(wid * RL, 8), RL)])


def _make_scatter_kernel(W):
    rows_per_tile = RL // NS

    @functools.partial(
        pl.kernel,
        out_type=jax.ShapeDtypeStruct((NC * RL, W), jnp.float32),
        mesh=_sc_mesh(),
        scratch_types=[
            pltpu.VMEM((NCHUNK, CH), jnp.int32),
            pltpu.VMEM((NCHUNK, CH), jnp.int32),
            pltpu.VMEM((KR * CH, W), jnp.float32),
            pltpu.VMEM((KR * CH, W), jnp.float32),
            pltpu.VMEM_SHARED((RL, W), jnp.float32),
            pltpu.SemaphoreType.DMA,
            pltpu.SemaphoreType.DMA,
            pltpu.SemaphoreType.DMA,
            pltpu.SemaphoreType.DMA,
        ],
        compiler_params=pltpu.CompilerParams(
            needs_layout_passes=False, use_tc_tiling_on_sc=False),
    )
    def k(table, src2d_hbm, dst2d_hbm, zeros_hbm, out, sb, db, r0, r1,
          acc, semg0, semg1, sems0, sems1):
        c = lax.axis_index("c")
        s = lax.axis_index("s")
        wid = c * NS + s
        pltpu.sync_copy(src2d_hbm.at[pl.ds(wid * NCHUNK, NCHUNK)], sb)
        pltpu.sync_copy(dst2d_hbm.at[pl.ds(wid * NCHUNK, NCHUNK)], db)
        zrow = pl.multiple_of(s * rows_per_tile, 8)
        pltpu.sync_copy(zeros_hbm.at[pl.ds(zrow, rows_per_tile)],
                        acc.at[pl.ds(zrow, rows_per_tile)])
        plsc.subcore_barrier()

        def gather(q, rbuf, semg):
            return pltpu.async_copy(table.at[sb.at[q]], rbuf, semg)

        def scatter(q, rbuf, sems):
            return pltpu.async_copy(rbuf, acc.at[db.at[q]], sems, add=True)

        def wait_g(q, rbuf, semg):
            pltpu.make_async_copy(table.at[sb.at[q]], rbuf, semg).wait()

        def wait_s(q, rbuf, sems):
            pltpu.make_async_copy(rbuf, acc.at[db.at[q]], sems).wait()

        gather(0, r0, semg0)

        bufs = ((r0, semg0, sems0), (r1, semg1, sems1))

        def pair(p, _):
            q0 = 2 * p
            for b in range(2):
                q = q0 + b
                rb, semg_b, sems_b = bufs[b]
                rn, semg_n, sems_n = bufs[1 - b]
                wait_g(q, rb, semg_b)
                scatter(q, rb, sems_b)

                @pl.when(q + 1 < SQ)
                def _():
                    @pl.when(q >= 1)
                    def _():
                        wait_s(q - 1, rn, sems_n)
                    gather(q + 1, rn, semg_n)
            return _

        lax.fori_loop(0, SQ // 2, pair, None)
        wait_s(SQ - 2, r0, sems0)
        wait_s(SQ - 1, r1, sems1)
        plsc.subcore_barrier()
        pltpu.sync_copy(acc.at[pl.ds(zrow, rows_per_tile)],
                        out.at[pl.ds(c * RL + zrow, rows_per_tile)])

    return k


@functools.lru_cache(maxsize=None)
def _sc_kernels():
    return _make_deg_kernel(), _make_scatter_kernel(64), _make_scatter_kernel(32)


def _deg_combine_body(parts_ref, out_ref):
    x = parts_ref[...]
    d0 = jnp.sum(x[:NS], axis=0, keepdims=True)
    d1 = jnp.sum(x[NS:], axis=0, keepdims=True)
    out_ref[...] = jax.lax.rsqrt(jnp.concatenate([d0, d1], axis=0) + 1.0)


BR = 2000
NBG = N // BR


def _t1_body(x_ref, w1_ref, dinv_ref, out_ref):
    out_ref[...] = dinv_ref[...] * _dot(x_ref[...], w1_ref[...])


def _mid_body(acc_ref, g_ref, b_ref, dinv_ref, w_ref, out_ref):
    y = jnp.maximum(dinv_ref[...] * (acc_ref[0] + g_ref[...]) + b_ref[...], 0.0)
    out_ref[...] = dinv_ref[...] * _dot(y, w_ref[...])


def _fin_body(acc_ref, g3_ref, dinv_ref, w3_ref, b3_ref, out_ref):
    z = dinv_ref[...] * (acc_ref[0] + g3_ref[...])
    out_ref[...] = _dot(z, w3_ref[...]) + b3_ref[...]


def _att_body(y3_ref, watt_ref, out_ref):
    y3 = y3_ref[0]
    m = jnp.mean(y3, axis=0, keepdims=True)
    cvec = jnp.tanh(_dot(m, watt_ref[...]))
    a = jax.nn.sigmoid(
        lax.dot_general(y3, cvec, (((1,), (1,)), ((), ())), precision=_HI))
    pooled = jnp.sum(a * y3, axis=0, keepdims=True)
    out_ref[...] = jnp.broadcast_to(pooled, (8, D))


def _head_body(hi_ref, hj_rep_ref, hcat_ref, wt_ref, sel_ref, vt_ref, bntn_ref,
               m1w_ref, m1b_ref, m2w_ref, m2b_ref, m3w_ref, m3b_ref,
               m4w_ref, m4b_ref, sw_ref, sb_ref, out_ref):
    u = _dot(hi_ref[...], wt_ref[...])
    bilinear = _dot(u * hj_rep_ref[...], sel_ref[...])
    lin = _dot(hcat_ref[...], vt_ref[...])
    inter = jnp.tanh(bilinear + lin + bntn_ref[...])
    inter = jnp.maximum(_dot(inter, m1w_ref[...]) + m1b_ref[...], 0.0)
    inter = jnp.maximum(_dot(inter, m2w_ref[...]) + m2b_ref[...], 0.0)
    inter = jnp.maximum(_dot(inter, m3w_ref[...]) + m3b_ref[...], 0.0)
    inter = jnp.maximum(_dot(inter, m4w_ref[...]) + m4b_ref[...], 0.0)
    out_ref[...] = jax.nn.sigmoid(_dot(inter, sw_ref[...]) + sb_ref[...])


def kernel(x_i, edge_index_i, x_j, edge_index_j, W1, b1, W2, b2, W3, b3,
           W_att, W_ntn, V_ntn, b_ntn, M1w, M1b, M2w, M2b, M3w, M3b,
           M4w, M4b, score_w, score_b):
    f32 = jnp.float32
    padg = EPG - E
    zpad = jnp.zeros((padg,), jnp.int32)
    tpad = jnp.full((padg,), TRASH, jnp.int32)
    src = jnp.concatenate([edge_index_i[0], zpad, edge_index_j[0] + N, zpad])
    dst = jnp.concatenate([edge_index_i[1], tpad, edge_index_j[1], tpad])
    x2 = jnp.concatenate([x_i, x_j], axis=0)
    src2d = src.reshape(EP // CH, CH)
    dst2d = dst.reshape(EP // CH, CH)

    deg_kernel, scatter64, scatter32 = _sc_kernels()

    deg_parts = deg_kernel(dst)
    dinv2 = pl.pallas_call(
        _deg_combine_body,
        out_shape=jax.ShapeDtypeStruct((2, RL), f32),
    )(deg_parts.reshape(NW, RL))
    dinv = jnp.concatenate([dinv2[0, :N], dinv2[1, :N]])[:, None]

    zeros64 = jnp.zeros((RL, 64), f32)
    zeros32 = jnp.zeros((RL, 32), f32)

    g1 = pl.pallas_call(
        _t1_body,
        grid=(2 * NBG,),
        in_specs=[
            pl.BlockSpec((BR, D), lambda r: (r, 0)),
            pl.BlockSpec((D, 64), lambda r: (0, 0)),
            pl.BlockSpec((BR, 1), lambda r: (r, 0)),
        ],
        out_specs=pl.BlockSpec((BR, 64), lambda r: (r, 0)),
        out_shape=jax.ShapeDtypeStruct((RT, 64), f32),
    )(x2, W1, dinv)
    acc1 = scatter64(g1, src2d, dst2d, zeros64).reshape(NC, RL, 64)

    def _mid_call(body, acc, g, b, w, win, wout):
        return pl.pallas_call(
            body,
            grid=(2, NBG),
            in_specs=[
                pl.BlockSpec((1, BR, win), lambda g_, r: (g_, r, 0)),
                pl.BlockSpec((BR, win), lambda g_, r: (g_ * NBG + r, 0)),
                pl.BlockSpec((win,), lambda g_, r: (0,)),
                pl.BlockSpec((BR, 1), lambda g_, r: (g_ * NBG + r, 0)),
                pl.BlockSpec((win, wout), lambda g_, r: (0, 0)),
            ],
            out_specs=pl.BlockSpec((BR, wout), lambda g_, r: (g_ * NBG + r, 0)),
            out_shape=jax.ShapeDtypeStruct((RT, wout), f32),
        )(acc, g, b, dinv, w)

    g2 = _mid_call(_mid_body, acc1, g1, b1, W2, 64, 32)
    acc2 = scatter32(g2, src2d, dst2d, zeros32).reshape(NC, RL, 32)

    g3 = _mid_call(_mid_body, acc2, g2, b2, jnp.eye(32, dtype=f32), 32, 32)
    acc3 = scatter32(g3, src2d, dst2d, zeros32).reshape(NC, RL, 32)

    y3 = pl.pallas_call(
        _fin_body,
        grid=(2, NBG),
        in_specs=[
            pl.BlockSpec((1, BR, 32), lambda g_, r: (g_, r, 0)),
            pl.BlockSpec((BR, 32), lambda g_, r: (g_ * NBG + r, 0)),
            pl.BlockSpec((BR, 1), lambda g_, r: (g_ * NBG + r, 0)),
            pl.BlockSpec((32, D), lambda g_, r: (0, 0)),
            pl.BlockSpec((D,), lambda g_, r: (0,)),
        ],
        out_specs=pl.BlockSpec((BR, D), lambda g_, r: (g_ * NBG + r, 0)),
        out_shape=jax.ShapeDtypeStruct((RT, D), f32),
    )(acc3, g3, dinv, W3, b3)

    pooled = pl.pallas_call(
        _att_body,
        grid=(2,),
        in_specs=[
            pl.BlockSpec((1, N, D), lambda g: (g, 0, 0)),
            pl.BlockSpec((D, D), lambda g: (0, 0)),
        ],
        out_specs=pl.BlockSpec((8, D), lambda g: (g, 0)),
        out_shape=jax.ShapeDtypeStruct((16, D), f32),
    )(y3.reshape(2, N, D), W_att)
    pooled = pooled[::8, :]

    hi = pooled[0:1, :]
    hj = pooled[1, :]
    hj_rep = jnp.tile(hj, 16)[None, :]
    hcat = jnp.concatenate([pooled[0], hj])[None, :]
    wt = jnp.transpose(W_ntn, (1, 0, 2)).reshape(D, 16 * D)
    sel = jnp.repeat(jnp.eye(16, dtype=f32), D, axis=0)
    vt = V_ntn.T

    score = pl.pallas_call(
        _head_body, out_shape=jax.ShapeDtypeStruct((1, 1), f32),
    )(hi, hj_rep, hcat, wt, sel, vt, b_ntn[None, :],
      M1w, M1b[None, :], M2w, M2b[None, :], M3w, M3b[None, :],
      M4w, M4b[None, :], score_w, score_b[None, :])
    return score[0]


# speedup vs baseline: 1.0745x; 1.0745x over previous
import functools

import jax
import jax.numpy as jnp
from jax import lax
from jax.experimental import pallas as pl
from jax.experimental.pallas import tpu as pltpu
from jax.experimental.pallas import tpu_sc as plsc

N = 10000
E = 320000
D = 128
RT = 2 * N
RL = 10240
TRASH = N
NC = 2
NS = 16
NW = NC * NS
CH = 128
KR = 1
NCHUNK = 160
SQ = NCHUNK // KR
PW = NCHUNK * CH
EPG = NS * PW
EP = NC * EPG
DC = 2560

_HI = jax.lax.Precision.HIGHEST


def _dot(a, b):
    return jnp.dot(a, b, precision=_HI)


def _sc_mesh():
    return plsc.VectorSubcoreMesh(
        core_axis_name="c", subcore_axis_name="s",
        num_cores=NC, num_subcores=NS)


def _make_deg_kernel():
    return functools.partial(
        pl.kernel,
        out_type=jax.ShapeDtypeStruct((NW * RL,), jnp.float32),
        mesh=_sc_mesh(),
        scratch_types=[
            pltpu.VMEM((DC,), jnp.int32),
            pltpu.VMEM((RL,), jnp.float32),
        ],
        compiler_params=pltpu.CompilerParams(needs_layout_passes=False),
    )(_deg_body)


def _deg_body(dst_hbm, out, dstb, degv):
    wid = lax.axis_index("c") * NS + lax.axis_index("s")
    zeros16 = jnp.zeros((16,), jnp.float32)
    ones16 = jnp.ones((16,), jnp.float32)

    def zbody(i, _):
        degv[pl.ds(pl.multiple_of(i * 16, 16), 16)] = zeros16
        return _

    lax.fori_loop(0, RL // 16, zbody, None)

    base = pl.multiple_of(wid * PW, 8)

    def chunk(jc, _):
        pltpu.sync_copy(dst_hbm.at[pl.ds(pl.multiple_of(base + jc * DC, 8), DC)], dstb)

        def inner(t, _):
            idx = dstb[pl.ds(pl.multiple_of(t * 16, 16), 16)]
            plsc.addupdate_scatter(degv, [idx], ones16)
            return _

        lax.fori_loop(0, DC // 16, inner, None)
        return _

    lax.fori_loop(0, PW // DC, chunk, None)
    pltpu.sync_copy(degv, out.at[pl.ds(pl.multiple_of(wid * RL, 8), RL)])


def _make_scatter_kernel(W):
    rows_per_tile = RL // NS

    @functools.partial(
        pl.kernel,
        out_type=jax.ShapeDtypeStruct((NC * RL, W), jnp.float32),
        mesh=_sc_mesh(),
        scratch_types=[
            pltpu.VMEM((NCHUNK, CH), jnp.int32),
            pltpu.VMEM((NCHUNK, CH), jnp.int32),
            pltpu.VMEM((KR * CH, W), jnp.float32),
            pltpu.VMEM((KR * CH, W), jnp.float32),
            pltpu.VMEM_SHARED((RL, W), jnp.float32),
            pltpu.SemaphoreType.DMA,
            pltpu.SemaphoreType.DMA,
            pltpu.SemaphoreType.DMA,
            pltpu.SemaphoreType.DMA,
        ],
        compiler_params=pltpu.CompilerParams(
            needs_layout_passes=False, use_tc_tiling_on_sc=False),
    )
    def k(table, src2d_hbm, dst2d_hbm, zeros_hbm, out, sb, db, r0, r1,
          acc, semg0, semg1, sems0, sems1):
        c = lax.axis_index("c")
        s = lax.axis_index("s")
        wid = c * NS + s
        pltpu.sync_copy(src2d_hbm.at[pl.ds(wid * NCHUNK, NCHUNK)], sb)
        pltpu.sync_copy(dst2d_hbm.at[pl.ds(wid * NCHUNK, NCHUNK)], db)
        zrow = pl.multiple_of(s * rows_per_tile, 8)
        pltpu.sync_copy(zeros_hbm.at[pl.ds(zrow, rows_per_tile)],
                        acc.at[pl.ds(zrow, rows_per_tile)])
        plsc.subcore_barrier()

        def gather(q, rbuf, semg):
            return pltpu.async_copy(table.at[sb.at[q]], rbuf, semg)

        def wait_g(q, rbuf, semg):
            pltpu.make_async_copy(table.at[sb.at[q]], rbuf, semg).wait()

        gather(0, r0, semg0)

        np2 = NCHUNK // 2

        def pair(p, _):
            j0 = 2 * p
            gather(j0 + 1, r1, semg1)
            wait_g(j0, r0, semg0)
            pltpu.sync_copy(r0, acc.at[db.at[j0]], add=True)

            @pl.when(p < np2 - 1)
            def _():
                gather(j0 + 2, r0, semg0)

            wait_g(j0 + 1, r1, semg1)
            pltpu.sync_copy(r1, acc.at[db.at[j0 + 1]], add=True)
            return _

        lax.fori_loop(0, np2, pair, None)
        plsc.subcore_barrier()
        pltpu.sync_copy(acc.at[pl.ds(zrow, rows_per_tile)],
                        out.at[pl.ds(c * RL + zrow, rows_per_tile)])

    return k


@functools.lru_cache(maxsize=None)
def _sc_kernels():
    return _make_deg_kernel(), _make_scatter_kernel(64), _make_scatter_kernel(32)


def _deg_combine_body(parts_ref, out_ref):
    x = parts_ref[...]
    d0 = jnp.sum(x[:NS], axis=0, keepdims=True)
    d1 = jnp.sum(x[NS:], axis=0, keepdims=True)
    out_ref[...] = jax.lax.rsqrt(jnp.concatenate([d0, d1], axis=0) + 1.0)


BR = 2000
NBG = N // BR


def _t1_body(x_ref, w1_ref, dinv_ref, out_ref):
    out_ref[...] = dinv_ref[...] * _dot(x_ref[...], w1_ref[...])


def _mid_body(acc_ref, g_ref, b_ref, dinv_ref, w_ref, out_ref):
    y = jnp.maximum(dinv_ref[...] * (acc_ref[0] + g_ref[...]) + b_ref[...], 0.0)
    out_ref[...] = dinv_ref[...] * _dot(y, w_ref[...])


def _fin_body(acc_ref, g3_ref, dinv_ref, w3_ref, b3_ref, out_ref):
    z = dinv_ref[...] * (acc_ref[0] + g3_ref[...])
    out_ref[...] = _dot(z, w3_ref[...]) + b3_ref[...]


def _att_body(y3_ref, watt_ref, out_ref):
    y3 = y3_ref[0]
    m = jnp.mean(y3, axis=0, keepdims=True)
    cvec = jnp.tanh(_dot(m, watt_ref[...]))
    a = jax.nn.sigmoid(
        lax.dot_general(y3, cvec, (((1,), (1,)), ((), ())), precision=_HI))
    pooled = jnp.sum(a * y3, axis=0, keepdims=True)
    out_ref[...] = jnp.broadcast_to(pooled, (8, D))


def _head_body(hi_ref, hj_rep_ref, hcat_ref, wt_ref, sel_ref, vt_ref, bntn_ref,
               m1w_ref, m1b_ref, m2w_ref, m2b_ref, m3w_ref, m3b_ref,
               m4w_ref, m4b_ref, sw_ref, sb_ref, out_ref):
    u = _dot(hi_ref[...], wt_ref[...])
    bilinear = _dot(u * hj_rep_ref[...], sel_ref[...])
    lin = _dot(hcat_ref[...], vt_ref[...])
    inter = jnp.tanh(bilinear + lin + bntn_ref[...])
    inter = jnp.maximum(_dot(inter, m1w_ref[...]) + m1b_ref[...], 0.0)
    inter = jnp.maximum(_dot(inter, m2w_ref[...]) + m2b_ref[...], 0.0)
    inter = jnp.maximum(_dot(inter, m3w_ref[...]) + m3b_ref[...], 0.0)
    inter = jnp.maximum(_dot(inter, m4w_ref[...]) + m4b_ref[...], 0.0)
    out_ref[...] = jax.nn.sigmoid(_dot(inter, sw_ref[...]) + sb_ref[...])


def kernel(x_i, edge_index_i, x_j, edge_index_j, W1, b1, W2, b2, W3, b3,
           W_att, W_ntn, V_ntn, b_ntn, M1w, M1b, M2w, M2b, M3w, M3b,
           M4w, M4b, score_w, score_b):
    f32 = jnp.float32
    padg = EPG - E
    zpad = jnp.zeros((padg,), jnp.int32)
    tpad = jnp.full((padg,), TRASH, jnp.int32)
    src = jnp.concatenate([edge_index_i[0], zpad, edge_index_j[0] + N, zpad])
    dst = jnp.concatenate([edge_index_i[1], tpad, edge_index_j[1], tpad])
    x2 = jnp.concatenate([x_i, x_j], axis=0)
    src2d = src.reshape(EP // CH, CH)
    dst2d = dst.reshape(EP // CH, CH)

    deg_kernel, scatter64, scatter32 = _sc_kernels()

    deg_parts = deg_kernel(dst)
    dinv2 = pl.pallas_call(
        _deg_combine_body,
        out_shape=jax.ShapeDtypeStruct((2, RL), f32),
    )(deg_parts.reshape(NW, RL))
    dinv = jnp.concatenate([dinv2[0, :N], dinv2[1, :N]])[:, None]

    zeros64 = jnp.zeros((RL, 64), f32)
    zeros32 = jnp.zeros((RL, 32), f32)

    g1 = pl.pallas_call(
        _t1_body,
        grid=(2 * NBG,),
        in_specs=[
            pl.BlockSpec((BR, D), lambda r: (r, 0)),
            pl.BlockSpec((D, 64), lambda r: (0, 0)),
            pl.BlockSpec((BR, 1), lambda r: (r, 0)),
        ],
        out_specs=pl.BlockSpec((BR, 64), lambda r: (r, 0)),
        out_shape=jax.ShapeDtypeStruct((RT, 64), f32),
    )(x2, W1, dinv)
    acc1 = scatter64(g1, src2d, dst2d, zeros64).reshape(NC, RL, 64)

    def _mid_call(body, acc, g, b, w, win, wout):
        return pl.pallas_call(
            body,
            grid=(2, NBG),
            in_specs=[
                pl.BlockSpec((1, BR, win), lambda g_, r: (g_, r, 0)),
                pl.BlockSpec((BR, win), lambda g_, r: (g_ * NBG + r, 0)),
                pl.BlockSpec((win,), lambda g_, r: (0,)),
                pl.BlockSpec((BR, 1), lambda g_, r: (g_ * NBG + r, 0)),
                pl.BlockSpec((win, wout), lambda g_, r: (0, 0)),
            ],
            out_specs=pl.BlockSpec((BR, wout), lambda g_, r: (g_ * NBG + r, 0)),
            out_shape=jax.ShapeDtypeStruct((RT, wout), f32),
        )(acc, g, b, dinv, w)

    g2 = _mid_call(_mid_body, acc1, g1, b1, W2, 64, 32)
    acc2 = scatter32(g2, src2d, dst2d, zeros32).reshape(NC, RL, 32)

    g3 = _mid_call(_mid_body, acc2, g2, b2, jnp.eye(32, dtype=f32), 32, 32)
    acc3 = scatter32(g3, src2d, dst2d, zeros32).reshape(NC, RL, 32)

    y3 = pl.pallas_call(
        _fin_body,
        grid=(2, NBG),
        in_specs=[
            pl.BlockSpec((1, BR, 32), lambda g_, r: (g_, r, 0)),
            pl.BlockSpec((BR, 32), lambda g_, r: (g_ * NBG + r, 0)),
            pl.BlockSpec((BR, 1), lambda g_, r: (g_ * NBG + r, 0)),
            pl.BlockSpec((32, D), lambda g_, r: (0, 0)),
            pl.BlockSpec((D,), lambda g_, r: (0,)),
        ],
        out_specs=pl.BlockSpec((BR, D), lambda g_, r: (g_ * NBG + r, 0)),
        out_shape=jax.ShapeDtypeStruct((RT, D), f32),
    )(acc3, g3, dinv, W3, b3)

    pooled = pl.pallas_call(
        _att_body,
        grid=(2,),
        in_specs=[
            pl.BlockSpec((1, N, D), lambda g: (g, 0, 0)),
            pl.BlockSpec((D, D), lambda g: (0, 0)),
        ],
        out_specs=pl.BlockSpec((8, D), lambda g: (g, 0)),
        out_shape=jax.ShapeDtypeStruct((16, D), f32),
    )(y3.reshape(2, N, D), W_att)
    pooled = pooled[::8, :]

    hi = pooled[0:1, :]
    hj = pooled[1, :]
    hj_rep = jnp.tile(hj, 16)[None, :]
    hcat = jnp.concatenate([pooled[0], hj])[None, :]
    wt = jnp.transpose(W_ntn, (1, 0, 2)).reshape(D, 16 * D)
    sel = jnp.repeat(jnp.eye(16, dtype=f32), D, axis=0)
    vt = V_ntn.T

    score = pl.pallas_call(
        _head_body, out_shape=jax.ShapeDtypeStruct((1, 1), f32),
    )(hi, hj_rep, hcat, wt, sel, vt, b_ntn[None, :],
      M1w, M1b[None, :], M2w, M2b[None, :], M3w, M3b[None, :],
      M4w, M4b[None, :], score_w, score_b[None, :])
    return score[0]


# speedup vs baseline: 1.0749x; 1.0003x over previous
import functools

import jax
import jax.numpy as jnp
from jax import lax
from jax.experimental import pallas as pl
from jax.experimental.pallas import tpu as pltpu
from jax.experimental.pallas import tpu_sc as plsc

N = 10000
E = 320000
D = 128
RT = 2 * N
RL = 10240
TRASH = N
NC = 2
NS = 16
NW = NC * NS
CH = 128
KR = 1
NCHUNK = 160
SQ = NCHUNK // KR
PW = NCHUNK * CH
EPG = NS * PW
EP = NC * EPG
DC = 2560

_HI = jax.lax.Precision.HIGHEST


def _dot(a, b):
    return jnp.dot(a, b, precision=_HI)


def _sc_mesh():
    return plsc.VectorSubcoreMesh(
        core_axis_name="c", subcore_axis_name="s",
        num_cores=NC, num_subcores=NS)


def _make_deg_kernel():
    return functools.partial(
        pl.kernel,
        out_type=jax.ShapeDtypeStruct((NW * RL,), jnp.float32),
        mesh=_sc_mesh(),
        scratch_types=[
            pltpu.VMEM((DC,), jnp.int32),
            pltpu.VMEM((RL,), jnp.float32),
        ],
        compiler_params=pltpu.CompilerParams(needs_layout_passes=False),
    )(_deg_body)


def _deg_body(dst_hbm, out, dstb, degv):
    wid = lax.axis_index("c") * NS + lax.axis_index("s")
    zeros16 = jnp.zeros((16,), jnp.float32)
    ones16 = jnp.ones((16,), jnp.float32)

    def zbody(i, _):
        degv[pl.ds(pl.multiple_of(i * 16, 16), 16)] = zeros16
        return _

    lax.fori_loop(0, RL // 16, zbody, None)

    base = pl.multiple_of(wid * PW, 8)

    def chunk(jc, _):
        pltpu.sync_copy(dst_hbm.at[pl.ds(pl.multiple_of(base + jc * DC, 8), DC)], dstb)

        def inner(t, _):
            idx = dstb[pl.ds(pl.multiple_of(t * 16, 16), 16)]
            plsc.addupdate_scatter(degv, [idx], ones16)
            return _

        lax.fori_loop(0, DC // 16, inner, None)
        return _

    lax.fori_loop(0, PW // DC, chunk, None)
    pltpu.sync_copy(degv, out.at[pl.ds(pl.multiple_of(wid * RL, 8), RL)])


def _make_scatter_kernel(W):
    rows_per_tile = RL // NS

    @functools.partial(
        pl.kernel,
        out_type=jax.ShapeDtypeStruct((NC * RL, W), jnp.float32),
        mesh=_sc_mesh(),
        scratch_types=[
            pltpu.VMEM((PW,), jnp.int32),
            pltpu.VMEM((NCHUNK, CH), jnp.int32),
            pltpu.VMEM((KR * CH, W), jnp.float32),
            pltpu.VMEM((KR * CH, W), jnp.float32),
            pltpu.VMEM_SHARED((RL, W), jnp.float32),
            pltpu.SemaphoreType.DMA,
            pltpu.SemaphoreType.DMA,
        ],
        compiler_params=pltpu.CompilerParams(
            needs_layout_passes=False, use_tc_tiling_on_sc=False),
    )
    def k(table, src_hbm, dst2d_hbm, zeros_hbm, out, sb, db, r0, r1,
          acc, semg0, semg1):
        c = lax.axis_index("c")
        s = lax.axis_index("s")
        wid = c * NS + s
        pltpu.sync_copy(src_hbm.at[pl.ds(pl.multiple_of(wid * PW, 8), PW)], sb)
        pltpu.sync_copy(dst2d_hbm.at[pl.ds(wid * NCHUNK, NCHUNK)], db)
        zrow = pl.multiple_of(s * rows_per_tile, 8)
        pltpu.sync_copy(zeros_hbm.at[pl.ds(zrow, rows_per_tile)],
                        acc.at[pl.ds(zrow, rows_per_tile)])
        plsc.subcore_barrier()

        def sidx(j):
            return sb.at[pl.ds(pl.multiple_of(j * CH, 8), CH)]

        def gather(q, rbuf, semg):
            return pltpu.async_copy(table.at[sidx(q)], rbuf, semg)

        def wait_g(q, rbuf, semg):
            pltpu.make_async_copy(table.at[sidx(q)], rbuf, semg).wait()

        gather(0, r0, semg0)

        np2 = NCHUNK // 2

        def pair(p, _):
            j0 = 2 * p
            gather(j0 + 1, r1, semg1)
            wait_g(j0, r0, semg0)
            pltpu.sync_copy(r0, acc.at[db.at[j0]], add=True)

            @pl.when(p < np2 - 1)
            def _():
                gather(j0 + 2, r0, semg0)

            wait_g(j0 + 1, r1, semg1)
            pltpu.sync_copy(r1, acc.at[db.at[j0 + 1]], add=True)
            return _

        lax.fori_loop(0, np2, pair, None)
        plsc.subcore_barrier()
        pltpu.sync_copy(acc.at[pl.ds(zrow, rows_per_tile)],
                        out.at[pl.ds(c * RL + zrow, rows_per_tile)])

    return k


@functools.lru_cache(maxsize=None)
def _sc_kernels():
    return _make_deg_kernel(), _make_scatter_kernel(64), _make_scatter_kernel(32)


def _deg_combine_body(parts_ref, out_ref):
    x = parts_ref[...]
    d0 = jnp.sum(x[:NS], axis=0, keepdims=True)
    d1 = jnp.sum(x[NS:], axis=0, keepdims=True)
    out_ref[...] = jax.lax.rsqrt(jnp.concatenate([d0, d1], axis=0) + 1.0)


BR = 2000
NBG = N // BR


def _t1_body(x_ref, w1_ref, dinv_ref, out_ref):
    out_ref[...] = dinv_ref[...] * _dot(x_ref[...], w1_ref[...])


def _mid_body(acc_ref, g_ref, b_ref, dinv_ref, w_ref, out_ref):
    y = jnp.maximum(dinv_ref[...] * (acc_ref[0] + g_ref[...]) + b_ref[...], 0.0)
    out_ref[...] = dinv_ref[...] * _dot(y, w_ref[...])


def _fin_body(acc_ref, g3_ref, dinv_ref, w3_ref, b3_ref, out_ref):
    z = dinv_ref[...] * (acc_ref[0] + g3_ref[...])
    out_ref[...] = _dot(z, w3_ref[...]) + b3_ref[...]


def _att_body(y3_ref, watt_ref, out_ref):
    y3 = y3_ref[0]
    m = jnp.mean(y3, axis=0, keepdims=True)
    cvec = jnp.tanh(_dot(m, watt_ref[...]))
    a = jax.nn.sigmoid(
        lax.dot_general(y3, cvec, (((1,), (1,)), ((), ())), precision=_HI))
    pooled = jnp.sum(a * y3, axis=0, keepdims=True)
    out_ref[...] = jnp.broadcast_to(pooled, (8, D))


def _head_body(hi_ref, hj_rep_ref, hcat_ref, wt_ref, sel_ref, vt_ref, bntn_ref,
               m1w_ref, m1b_ref, m2w_ref, m2b_ref, m3w_ref, m3b_ref,
               m4w_ref, m4b_ref, sw_ref, sb_ref, out_ref):
    u = _dot(hi_ref[...], wt_ref[...])
    bilinear = _dot(u * hj_rep_ref[...], sel_ref[...])
    lin = _dot(hcat_ref[...], vt_ref[...])
    inter = jnp.tanh(bilinear + lin + bntn_ref[...])
    inter = jnp.maximum(_dot(inter, m1w_ref[...]) + m1b_ref[...], 0.0)
    inter = jnp.maximum(_dot(inter, m2w_ref[...]) + m2b_ref[...], 0.0)
    inter = jnp.maximum(_dot(inter, m3w_ref[...]) + m3b_ref[...], 0.0)
    inter = jnp.maximum(_dot(inter, m4w_ref[...]) + m4b_ref[...], 0.0)
    out_ref[...] = jax.nn.sigmoid(_dot(inter, sw_ref[...]) + sb_ref[...])


def kernel(x_i, edge_index_i, x_j, edge_index_j, W1, b1, W2, b2, W3, b3,
           W_att, W_ntn, V_ntn, b_ntn, M1w, M1b, M2w, M2b, M3w, M3b,
           M4w, M4b, score_w, score_b):
    f32 = jnp.float32
    padg = EPG - E
    zpad = jnp.zeros((padg,), jnp.int32)
    tpad = jnp.full((padg,), TRASH, jnp.int32)
    src = jnp.concatenate([edge_index_i[0], zpad, edge_index_j[0] + N, zpad])
    dst = jnp.concatenate([edge_index_i[1], tpad, edge_index_j[1], tpad])
    x2 = jnp.concatenate([x_i, x_j], axis=0)
    dst2d = dst.reshape(EP // CH, CH)

    deg_kernel, scatter64, scatter32 = _sc_kernels()

    deg_parts = deg_kernel(dst)
    dinv2 = pl.pallas_call(
        _deg_combine_body,
        out_shape=jax.ShapeDtypeStruct((2, RL), f32),
    )(deg_parts.reshape(NW, RL))
    dinv = jnp.concatenate([dinv2[0, :N], dinv2[1, :N]])[:, None]

    zeros64 = jnp.zeros((RL, 64), f32)
    zeros32 = jnp.zeros((RL, 32), f32)

    g1 = pl.pallas_call(
        _t1_body,
        grid=(2 * NBG,),
        in_specs=[
            pl.BlockSpec((BR, D), lambda r: (r, 0)),
            pl.BlockSpec((D, 64), lambda r: (0, 0)),
            pl.BlockSpec((BR, 1), lambda r: (r, 0)),
        ],
        out_specs=pl.BlockSpec((BR, 64), lambda r: (r, 0)),
        out_shape=jax.ShapeDtypeStruct((RT, 64), f32),
    )(x2, W1, dinv)
    acc1 = scatter64(g1, src, dst2d, zeros64).reshape(NC, RL, 64)

    def _mid_call(body, acc, g, b, w, win, wout):
        return pl.pallas_call(
            body,
            grid=(2, NBG),
            in_specs=[
                pl.BlockSpec((1, BR, win), lambda g_, r: (g_, r, 0)),
                pl.BlockSpec((BR, win), lambda g_, r: (g_ * NBG + r, 0)),
                pl.BlockSpec((win,), lambda g_, r: (0,)),
                pl.BlockSpec((BR, 1), lambda g_, r: (g_ * NBG + r, 0)),
                pl.BlockSpec((win, wout), lambda g_, r: (0, 0)),
            ],
            out_specs=pl.BlockSpec((BR, wout), lambda g_, r: (g_ * NBG + r, 0)),
            out_shape=jax.ShapeDtypeStruct((RT, wout), f32),
        )(acc, g, b, dinv, w)

    g2 = _mid_call(_mid_body, acc1, g1, b1, W2, 64, 32)
    acc2 = scatter32(g2, src, dst2d, zeros32).reshape(NC, RL, 32)

    g3 = _mid_call(_mid_body, acc2, g2, b2, jnp.eye(32, dtype=f32), 32, 32)
    acc3 = scatter32(g3, src, dst2d, zeros32).reshape(NC, RL, 32)

    y3 = pl.pallas_call(
        _fin_body,
        grid=(2, NBG),
        in_specs=[
            pl.BlockSpec((1, BR, 32), lambda g_, r: (g_, r, 0)),
            pl.BlockSpec((BR, 32), lambda g_, r: (g_ * NBG + r, 0)),
            pl.BlockSpec((BR, 1), lambda g_, r: (g_ * NBG + r, 0)),
            pl.BlockSpec((32, D), lambda g_, r: (0, 0)),
            pl.BlockSpec((D,), lambda g_, r: (0,)),
        ],
        out_specs=pl.BlockSpec((BR, D), lambda g_, r: (g_ * NBG + r, 0)),
        out_shape=jax.ShapeDtypeStruct((RT, D), f32),
    )(acc3, g3, dinv, W3, b3)

    pooled = pl.pallas_call(
        _att_body,
        grid=(2,),
        in_specs=[
            pl.BlockSpec((1, N, D), lambda g: (g, 0, 0)),
            pl.BlockSpec((D, D), lambda g: (0, 0)),
        ],
        out_specs=pl.BlockSpec((8, D), lambda g: (g, 0)),
        out_shape=jax.ShapeDtypeStruct((16, D), f32),
    )(y3.reshape(2, N, D), W_att)
    pooled = pooled[::8, :]

    hi = pooled[0:1, :]
    hj = pooled[1, :]
    hj_rep = jnp.tile(hj, 16)[None, :]
    hcat = jnp.concatenate([pooled[0], hj])[None, :]
    wt = jnp.transpose(W_ntn, (1, 0, 2)).reshape(D, 16 * D)
    sel = jnp.repeat(jnp.eye(16, dtype=f32), D, axis=0)
    vt = V_ntn.T

    score = pl.pallas_call(
        _head_body, out_shape=jax.ShapeDtypeStruct((1, 1), f32),
    )(hi, hj_rep, hcat, wt, sel, vt, b_ntn[None, :],
      M1w, M1b[None, :], M2w, M2b[None, :], M3w, M3b[None, :],
      M4w, M4b[None, :], score_w, score_b[None, :])
    return score[0]


# speedup vs baseline: 1.4745x; 1.3718x over previous
import functools

import jax
import jax.numpy as jnp
from jax import lax
from jax.experimental import pallas as pl
from jax.experimental.pallas import tpu as pltpu
from jax.experimental.pallas import tpu_sc as plsc

N = 10000
E = 320000
D = 128
RT = 2 * N
RL = 10240
TRASH = N
NC = 2
NS = 16
NW = NC * NS
CH = 128
KR = 1
NCHUNK = 158
SQ = NCHUNK // KR
PW = NCHUNK * CH
EPG = NS * PW
EP = NC * EPG
DC = 2528

_HI = jax.lax.Precision.HIGHEST


def _dot(a, b):
    return jnp.dot(a, b, precision=_HI)


def _sc_mesh():
    return plsc.VectorSubcoreMesh(
        core_axis_name="c", subcore_axis_name="s",
        num_cores=NC, num_subcores=NS)


def _make_deg_kernel():
    return functools.partial(
        pl.kernel,
        out_type=jax.ShapeDtypeStruct((NW * RL,), jnp.float32),
        mesh=_sc_mesh(),
        scratch_types=[
            pltpu.VMEM((DC,), jnp.int32),
            pltpu.VMEM((RL,), jnp.float32),
        ],
        compiler_params=pltpu.CompilerParams(needs_layout_passes=False),
    )(_deg_body)


def _deg_body(dst_hbm, out, dstb, degv):
    wid = lax.axis_index("c") * NS + lax.axis_index("s")
    zeros16 = jnp.zeros((16,), jnp.float32)
    ones16 = jnp.ones((16,), jnp.float32)

    def zbody(i, _):
        degv[pl.ds(pl.multiple_of(i * 16, 16), 16)] = zeros16
        return _

    lax.fori_loop(0, RL // 16, zbody, None)

    base = pl.multiple_of(wid * PW, 8)

    def chunk(jc, _):
        pltpu.sync_copy(dst_hbm.at[pl.ds(pl.multiple_of(base + jc * DC, 8), DC)], dstb)

        def inner(t, _):
            idx = dstb[pl.ds(pl.multiple_of(t * 16, 16), 16)]
            plsc.addupdate_scatter(degv, [idx], ones16)
            return _

        lax.fori_loop(0, DC // 16, inner, None)
        return _

    lax.fori_loop(0, PW // DC, chunk, None)
    pltpu.sync_copy(degv, out.at[pl.ds(pl.multiple_of(wid * RL, 8), RL)])


def _make_scatter_kernel(W):
    rows_per_tile = RL // NS

    @functools.partial(
        pl.kernel,
        out_type=jax.ShapeDtypeStruct((NC * RL, W), jnp.float32),
        mesh=_sc_mesh(),
        scratch_types=[
            pltpu.VMEM((PW,), jnp.int32),
            pltpu.VMEM((NCHUNK, CH), jnp.int32),
            pltpu.VMEM((KR * CH, W), jnp.float32),
            pltpu.VMEM((KR * CH, W), jnp.float32),
            pltpu.VMEM_SHARED((RL, W), jnp.float32),
            pltpu.SemaphoreType.DMA,
            pltpu.SemaphoreType.DMA,
        ],
        compiler_params=pltpu.CompilerParams(
            needs_layout_passes=False, use_tc_tiling_on_sc=False),
    )
    def k(table, src_hbm, dst2d_hbm, zeros_hbm, out, sb, db, r0, r1,
          acc, semg0, semg1):
        c = lax.axis_index("c")
        s = lax.axis_index("s")
        wid = c * NS + s
        pltpu.sync_copy(src_hbm.at[pl.ds(pl.multiple_of(wid * PW, 8), PW)], sb)
        pltpu.sync_copy(dst2d_hbm.at[pl.ds(wid * NCHUNK, NCHUNK)], db)
        zrow = pl.multiple_of(s * rows_per_tile, 8)
        pltpu.sync_copy(zeros_hbm.at[pl.ds(zrow, rows_per_tile)],
                        acc.at[pl.ds(zrow, rows_per_tile)])
        plsc.subcore_barrier()

        def sidx(j):
            return sb.at[pl.ds(pl.multiple_of(j * CH, 8), CH)]

        def gather(q, rbuf, semg):
            return pltpu.async_copy(table.at[sidx(q)], rbuf, semg)

        def wait_g(q, rbuf, semg):
            pltpu.make_async_copy(table.at[sidx(q)], rbuf, semg).wait()

        gather(0, r0, semg0)

        np2 = NCHUNK // 2

        def pair(p, _):
            j0 = 2 * p
            gather(j0 + 1, r1, semg1)
            wait_g(j0, r0, semg0)
            pltpu.sync_copy(r0, acc.at[db.at[j0]], add=True)

            @pl.when(p < np2 - 1)
            def _():
                gather(j0 + 2, r0, semg0)

            wait_g(j0 + 1, r1, semg1)
            pltpu.sync_copy(r1, acc.at[db.at[j0 + 1]], add=True)
            return _

        lax.fori_loop(0, np2, pair, None)
        plsc.subcore_barrier()
        pltpu.sync_copy(acc.at[pl.ds(zrow, rows_per_tile)],
                        out.at[pl.ds(c * RL + zrow, rows_per_tile)])

    return k


@functools.lru_cache(maxsize=None)
def _sc_kernels():
    return _make_deg_kernel(), _make_scatter_kernel(64), _make_scatter_kernel(32)


def _deg_combine_body(parts_ref, out_ref):
    x = parts_ref[...]
    d0 = jnp.sum(x[:NS], axis=0, keepdims=True)
    d1 = jnp.sum(x[NS:], axis=0, keepdims=True)
    out_ref[...] = jax.lax.rsqrt(jnp.concatenate([d0, d1], axis=0) + 1.0)


BR = 2000
NBG = N // BR


def _t1_body(x_ref, w1_ref, dinv_ref, out_ref):
    out_ref[...] = dinv_ref[...] * _dot(x_ref[...], w1_ref[...])


def _mid_body(acc_ref, g_ref, b_ref, dinv_ref, w_ref, out_ref):
    y = jnp.maximum(dinv_ref[...] * (acc_ref[0] + g_ref[...]) + b_ref[...], 0.0)
    out_ref[...] = dinv_ref[...] * _dot(y, w_ref[...])


def _fin_body(acc_ref, g3_ref, dinv_ref, w3_ref, b3_ref, out_ref):
    z = dinv_ref[...] * (acc_ref[0] + g3_ref[...])
    out_ref[...] = _dot(z, w3_ref[...]) + b3_ref[...]


def _att_body(y3_ref, watt_ref, out_ref):
    y3 = y3_ref[0]
    m = jnp.mean(y3, axis=0, keepdims=True)
    cvec = jnp.tanh(_dot(m, watt_ref[...]))
    a = jax.nn.sigmoid(
        lax.dot_general(y3, cvec, (((1,), (1,)), ((), ())), precision=_HI))
    pooled = jnp.sum(a * y3, axis=0, keepdims=True)
    out_ref[...] = jnp.broadcast_to(pooled, (8, D))


def _head_body(hi_ref, hj_rep_ref, hcat_ref, wt_ref, sel_ref, vt_ref, bntn_ref,
               m1w_ref, m1b_ref, m2w_ref, m2b_ref, m3w_ref, m3b_ref,
               m4w_ref, m4b_ref, sw_ref, sb_ref, out_ref):
    u = _dot(hi_ref[...], wt_ref[...])
    bilinear = _dot(u * hj_rep_ref[...], sel_ref[...])
    lin = _dot(hcat_ref[...], vt_ref[...])
    inter = jnp.tanh(bilinear + lin + bntn_ref[...])
    inter = jnp.maximum(_dot(inter, m1w_ref[...]) + m1b_ref[...], 0.0)
    inter = jnp.maximum(_dot(inter, m2w_ref[...]) + m2b_ref[...], 0.0)
    inter = jnp.maximum(_dot(inter, m3w_ref[...]) + m3b_ref[...], 0.0)
    inter = jnp.maximum(_dot(inter, m4w_ref[...]) + m4b_ref[...], 0.0)
    out_ref[...] = jax.nn.sigmoid(_dot(inter, sw_ref[...]) + sb_ref[...])


def kernel(x_i, edge_index_i, x_j, edge_index_j, W1, b1, W2, b2, W3, b3,
           W_att, W_ntn, V_ntn, b_ntn, M1w, M1b, M2w, M2b, M3w, M3b,
           M4w, M4b, score_w, score_b):
    f32 = jnp.float32
    padg = EPG - E
    zpad = jnp.zeros((padg,), jnp.int32)
    tpad = TRASH + jnp.arange(padg, dtype=jnp.int32) % (RL - N)
    src = jnp.concatenate([edge_index_i[0], zpad, edge_index_j[0] + N, zpad])
    dst = jnp.concatenate([edge_index_i[1], tpad, edge_index_j[1], tpad])
    x2 = jnp.concatenate([x_i, x_j], axis=0)
    dst2d = dst.reshape(EP // CH, CH)

    deg_kernel, scatter64, scatter32 = _sc_kernels()

    deg_parts = deg_kernel(dst)
    dinv2 = pl.pallas_call(
        _deg_combine_body,
        out_shape=jax.ShapeDtypeStruct((2, RL), f32),
    )(deg_parts.reshape(NW, RL))
    dinv = jnp.concatenate([dinv2[0, :N], dinv2[1, :N]])[:, None]

    zeros64 = jnp.zeros((RL, 64), f32)
    zeros32 = jnp.zeros((RL, 32), f32)

    g1 = pl.pallas_call(
        _t1_body,
        grid=(2 * NBG,),
        in_specs=[
            pl.BlockSpec((BR, D), lambda r: (r, 0)),
            pl.BlockSpec((D, 64), lambda r: (0, 0)),
            pl.BlockSpec((BR, 1), lambda r: (r, 0)),
        ],
        out_specs=pl.BlockSpec((BR, 64), lambda r: (r, 0)),
        out_shape=jax.ShapeDtypeStruct((RT, 64), f32),
    )(x2, W1, dinv)
    acc1 = scatter64(g1, src, dst2d, zeros64).reshape(NC, RL, 64)

    def _mid_call(body, acc, g, b, w, win, wout):
        return pl.pallas_call(
            body,
            grid=(2, NBG),
            in_specs=[
                pl.BlockSpec((1, BR, win), lambda g_, r: (g_, r, 0)),
                pl.BlockSpec((BR, win), lambda g_, r: (g_ * NBG + r, 0)),
                pl.BlockSpec((win,), lambda g_, r: (0,)),
                pl.BlockSpec((BR, 1), lambda g_, r: (g_ * NBG + r, 0)),
                pl.BlockSpec((win, wout), lambda g_, r: (0, 0)),
            ],
            out_specs=pl.BlockSpec((BR, wout), lambda g_, r: (g_ * NBG + r, 0)),
            out_shape=jax.ShapeDtypeStruct((RT, wout), f32),
        )(acc, g, b, dinv, w)

    g2 = _mid_call(_mid_body, acc1, g1, b1, W2, 64, 32)
    acc2 = scatter32(g2, src, dst2d, zeros32).reshape(NC, RL, 32)

    g3 = _mid_call(_mid_body, acc2, g2, b2, jnp.eye(32, dtype=f32), 32, 32)
    acc3 = scatter32(g3, src, dst2d, zeros32).reshape(NC, RL, 32)

    y3 = pl.pallas_call(
        _fin_body,
        grid=(2, NBG),
        in_specs=[
            pl.BlockSpec((1, BR, 32), lambda g_, r: (g_, r, 0)),
            pl.BlockSpec((BR, 32), lambda g_, r: (g_ * NBG + r, 0)),
            pl.BlockSpec((BR, 1), lambda g_, r: (g_ * NBG + r, 0)),
            pl.BlockSpec((32, D), lambda g_, r: (0, 0)),
            pl.BlockSpec((D,), lambda g_, r: (0,)),
        ],
        out_specs=pl.BlockSpec((BR, D), lambda g_, r: (g_ * NBG + r, 0)),
        out_shape=jax.ShapeDtypeStruct((RT, D), f32),
    )(acc3, g3, dinv, W3, b3)

    pooled = pl.pallas_call(
        _att_body,
        grid=(2,),
        in_specs=[
            pl.BlockSpec((1, N, D), lambda g: (g, 0, 0)),
            pl.BlockSpec((D, D), lambda g: (0, 0)),
        ],
        out_specs=pl.BlockSpec((8, D), lambda g: (g, 0)),
        out_shape=jax.ShapeDtypeStruct((16, D), f32),
    )(y3.reshape(2, N, D), W_att)
    pooled = pooled[::8, :]

    hi = pooled[0:1, :]
    hj = pooled[1, :]
    hj_rep = jnp.tile(hj, 16)[None, :]
    hcat = jnp.concatenate([pooled[0], hj])[None, :]
    wt = jnp.transpose(W_ntn, (1, 0, 2)).reshape(D, 16 * D)
    sel = jnp.repeat(jnp.eye(16, dtype=f32), D, axis=0)
    vt = V_ntn.T

    score = pl.pallas_call(
        _head_body, out_shape=jax.ShapeDtypeStruct((1, 1), f32),
    )(hi, hj_rep, hcat, wt, sel, vt, b_ntn[None, :],
      M1w, M1b[None, :], M2w, M2b[None, :], M3w, M3b[None, :],
      M4w, M4b[None, :], score_w, score_b[None, :])
    return score[0]


# speedup vs baseline: 1.9400x; 1.3157x over previous
import functools

import jax
import jax.numpy as jnp
from jax import lax
from jax.experimental import pallas as pl
from jax.experimental.pallas import tpu as pltpu
from jax.experimental.pallas import tpu_sc as plsc

N = 10000
E = 320000
D = 128
RT = 2 * N
RL = 10240
TRASH = N
NC = 2
NS = 16
NW = NC * NS
CH = 128
KR = 1
NCHUNK = 158
SQ = NCHUNK // KR
PW = NCHUNK * CH
EPG = NS * PW
EP = NC * EPG
DC = 2528

_HI = jax.lax.Precision.HIGHEST


def _dot(a, b):
    return jnp.dot(a, b, precision=_HI)


def _sc_mesh():
    return plsc.VectorSubcoreMesh(
        core_axis_name="c", subcore_axis_name="s",
        num_cores=NC, num_subcores=NS)


def _make_deg_kernel():
    return functools.partial(
        pl.kernel,
        out_type=jax.ShapeDtypeStruct((NW * RL,), jnp.float32),
        mesh=_sc_mesh(),
        scratch_types=[
            pltpu.VMEM((DC,), jnp.int32),
            pltpu.VMEM((RL,), jnp.float32),
        ],
        compiler_params=pltpu.CompilerParams(needs_layout_passes=False),
    )(_deg_body)


def _deg_body(dst_hbm, out, dstb, degv):
    wid = lax.axis_index("c") * NS + lax.axis_index("s")
    zeros16 = jnp.zeros((16,), jnp.float32)
    ones16 = jnp.ones((16,), jnp.float32)

    def zbody(i, _):
        degv[pl.ds(pl.multiple_of(i * 16, 16), 16)] = zeros16
        return _

    lax.fori_loop(0, RL // 16, zbody, None)

    base = pl.multiple_of(wid * PW, 8)

    def chunk(jc, _):
        pltpu.sync_copy(dst_hbm.at[pl.ds(pl.multiple_of(base + jc * DC, 8), DC)], dstb)

        def inner(t, _):
            idx = dstb[pl.ds(pl.multiple_of(t * 16, 16), 16)]
            plsc.addupdate_scatter(degv, [idx], ones16)
            return _

        lax.fori_loop(0, DC // 16, inner, None)
        return _

    lax.fori_loop(0, PW // DC, chunk, None)
    pltpu.sync_copy(degv, out.at[pl.ds(pl.multiple_of(wid * RL, 8), RL)])


def _make_scatter_kernel(W):
    rows_per_tile = RL // NS

    @functools.partial(
        pl.kernel,
        out_type=jax.ShapeDtypeStruct((NC * RL, W), jnp.bfloat16),
        mesh=_sc_mesh(),
        scratch_types=[
            pltpu.VMEM((PW,), jnp.int32),
            pltpu.VMEM((NCHUNK, CH), jnp.int32),
            pltpu.VMEM((KR * CH, W), jnp.bfloat16),
            pltpu.VMEM((KR * CH, W), jnp.bfloat16),
            pltpu.VMEM_SHARED((RL, W), jnp.bfloat16),
            pltpu.SemaphoreType.DMA,
            pltpu.SemaphoreType.DMA,
        ],
        compiler_params=pltpu.CompilerParams(
            needs_layout_passes=False, use_tc_tiling_on_sc=False),
    )
    def k(table, src_hbm, dst2d_hbm, zeros_hbm, out, sb, db, r0, r1,
          acc, semg0, semg1):
        c = lax.axis_index("c")
        s = lax.axis_index("s")
        wid = c * NS + s
        pltpu.sync_copy(src_hbm.at[pl.ds(pl.multiple_of(wid * PW, 8), PW)], sb)
        pltpu.sync_copy(dst2d_hbm.at[pl.ds(wid * NCHUNK, NCHUNK)], db)
        zrow = pl.multiple_of(s * rows_per_tile, 8)
        pltpu.sync_copy(zeros_hbm.at[pl.ds(zrow, rows_per_tile)],
                        acc.at[pl.ds(zrow, rows_per_tile)])
        plsc.subcore_barrier()

        def sidx(j):
            return sb.at[pl.ds(pl.multiple_of(j * CH, 8), CH)]

        def gather(q, rbuf, semg):
            return pltpu.async_copy(table.at[sidx(q)], rbuf, semg)

        def wait_g(q, rbuf, semg):
            pltpu.make_async_copy(table.at[sidx(q)], rbuf, semg).wait()

        gather(0, r0, semg0)

        np2 = NCHUNK // 2

        def pair(p, _):
            j0 = 2 * p
            gather(j0 + 1, r1, semg1)
            wait_g(j0, r0, semg0)
            pltpu.sync_copy(r0, acc.at[db.at[j0]], add=True)

            @pl.when(p < np2 - 1)
            def _():
                gather(j0 + 2, r0, semg0)

            wait_g(j0 + 1, r1, semg1)
            pltpu.sync_copy(r1, acc.at[db.at[j0 + 1]], add=True)
            return _

        lax.fori_loop(0, np2, pair, None)
        plsc.subcore_barrier()
        pltpu.sync_copy(acc.at[pl.ds(zrow, rows_per_tile)],
                        out.at[pl.ds(c * RL + zrow, rows_per_tile)])

    return k


@functools.lru_cache(maxsize=None)
def _sc_kernels():
    return _make_deg_kernel(), _make_scatter_kernel(64), _make_scatter_kernel(32)


def _deg_combine_body(parts_ref, out_ref):
    x = parts_ref[...]
    d0 = jnp.sum(x[:NS], axis=0, keepdims=True)
    d1 = jnp.sum(x[NS:], axis=0, keepdims=True)
    out_ref[...] = jax.lax.rsqrt(jnp.concatenate([d0, d1], axis=0) + 1.0)


BR = 2000
NBG = N // BR


def _t1_body(x_ref, w1_ref, dinv_ref, out_ref):
    out_ref[...] = (dinv_ref[...] * _dot(x_ref[...], w1_ref[...])
                    ).astype(jnp.bfloat16)


def _mid_body(acc_ref, g_ref, b_ref, dinv_ref, w_ref, out_ref):
    accsum = acc_ref[0].astype(jnp.float32) + g_ref[...].astype(jnp.float32)
    y = jnp.maximum(dinv_ref[...] * accsum + b_ref[...], 0.0)
    out_ref[...] = (dinv_ref[...] * _dot(y, w_ref[...])).astype(jnp.bfloat16)


def _fin_body(acc_ref, g3_ref, dinv_ref, w3_ref, b3_ref, out_ref):
    accsum = acc_ref[0].astype(jnp.float32) + g3_ref[...].astype(jnp.float32)
    z = dinv_ref[...] * accsum
    out_ref[...] = _dot(z, w3_ref[...]) + b3_ref[...]


def _att_body(y3_ref, watt_ref, out_ref):
    y3 = y3_ref[0]
    m = jnp.mean(y3, axis=0, keepdims=True)
    cvec = jnp.tanh(_dot(m, watt_ref[...]))
    a = jax.nn.sigmoid(
        lax.dot_general(y3, cvec, (((1,), (1,)), ((), ())), precision=_HI))
    pooled = jnp.sum(a * y3, axis=0, keepdims=True)
    out_ref[...] = jnp.broadcast_to(pooled, (8, D))


def _head_body(hi_ref, hj_rep_ref, hcat_ref, wt_ref, sel_ref, vt_ref, bntn_ref,
               m1w_ref, m1b_ref, m2w_ref, m2b_ref, m3w_ref, m3b_ref,
               m4w_ref, m4b_ref, sw_ref, sb_ref, out_ref):
    u = _dot(hi_ref[...], wt_ref[...])
    bilinear = _dot(u * hj_rep_ref[...], sel_ref[...])
    lin = _dot(hcat_ref[...], vt_ref[...])
    inter = jnp.tanh(bilinear + lin + bntn_ref[...])
    inter = jnp.maximum(_dot(inter, m1w_ref[...]) + m1b_ref[...], 0.0)
    inter = jnp.maximum(_dot(inter, m2w_ref[...]) + m2b_ref[...], 0.0)
    inter = jnp.maximum(_dot(inter, m3w_ref[...]) + m3b_ref[...], 0.0)
    inter = jnp.maximum(_dot(inter, m4w_ref[...]) + m4b_ref[...], 0.0)
    out_ref[...] = jax.nn.sigmoid(_dot(inter, sw_ref[...]) + sb_ref[...])


def kernel(x_i, edge_index_i, x_j, edge_index_j, W1, b1, W2, b2, W3, b3,
           W_att, W_ntn, V_ntn, b_ntn, M1w, M1b, M2w, M2b, M3w, M3b,
           M4w, M4b, score_w, score_b):
    f32 = jnp.float32
    padg = EPG - E
    zpad = jnp.zeros((padg,), jnp.int32)
    tpad = TRASH + jnp.arange(padg, dtype=jnp.int32) % (RL - N)
    src = jnp.concatenate([edge_index_i[0], zpad, edge_index_j[0] + N, zpad])
    dst = jnp.concatenate([edge_index_i[1], tpad, edge_index_j[1], tpad])
    x2 = jnp.concatenate([x_i, x_j], axis=0)
    dst2d = dst.reshape(EP // CH, CH)

    deg_kernel, scatter64, scatter32 = _sc_kernels()

    deg_parts = deg_kernel(dst)
    dinv2 = pl.pallas_call(
        _deg_combine_body,
        out_shape=jax.ShapeDtypeStruct((2, RL), f32),
    )(deg_parts.reshape(NW, RL))
    dinv = jnp.concatenate([dinv2[0, :N], dinv2[1, :N]])[:, None]

    bf16 = jnp.bfloat16
    zeros64 = jnp.zeros((RL, 64), bf16)
    zeros32 = jnp.zeros((RL, 32), bf16)

    g1 = pl.pallas_call(
        _t1_body,
        grid=(2 * NBG,),
        in_specs=[
            pl.BlockSpec((BR, D), lambda r: (r, 0)),
            pl.BlockSpec((D, 64), lambda r: (0, 0)),
            pl.BlockSpec((BR, 1), lambda r: (r, 0)),
        ],
        out_specs=pl.BlockSpec((BR, 64), lambda r: (r, 0)),
        out_shape=jax.ShapeDtypeStruct((RT, 64), jnp.bfloat16),
    )(x2, W1, dinv)
    acc1 = scatter64(g1, src, dst2d, zeros64).reshape(NC, RL, 64)

    def _mid_call(body, acc, g, b, w, win, wout):
        return pl.pallas_call(
            body,
            grid=(2, NBG),
            in_specs=[
                pl.BlockSpec((1, BR, win), lambda g_, r: (g_, r, 0)),
                pl.BlockSpec((BR, win), lambda g_, r: (g_ * NBG + r, 0)),
                pl.BlockSpec((win,), lambda g_, r: (0,)),
                pl.BlockSpec((BR, 1), lambda g_, r: (g_ * NBG + r, 0)),
                pl.BlockSpec((win, wout), lambda g_, r: (0, 0)),
            ],
            out_specs=pl.BlockSpec((BR, wout), lambda g_, r: (g_ * NBG + r, 0)),
            out_shape=jax.ShapeDtypeStruct((RT, wout), jnp.bfloat16),
        )(acc, g, b, dinv, w)

    g2 = _mid_call(_mid_body, acc1, g1, b1, W2, 64, 32)
    acc2 = scatter32(g2, src, dst2d, zeros32).reshape(NC, RL, 32)

    g3 = _mid_call(_mid_body, acc2, g2, b2, jnp.eye(32, dtype=f32), 32, 32)
    acc3 = scatter32(g3, src, dst2d, zeros32).reshape(NC, RL, 32)

    y3 = pl.pallas_call(
        _fin_body,
        grid=(2, NBG),
        in_specs=[
            pl.BlockSpec((1, BR, 32), lambda g_, r: (g_, r, 0)),
            pl.BlockSpec((BR, 32), lambda g_, r: (g_ * NBG + r, 0)),
            pl.BlockSpec((BR, 1), lambda g_, r: (g_ * NBG + r, 0)),
            pl.BlockSpec((32, D), lambda g_, r: (0, 0)),
            pl.BlockSpec((D,), lambda g_, r: (0,)),
        ],
        out_specs=pl.BlockSpec((BR, D), lambda g_, r: (g_ * NBG + r, 0)),
        out_shape=jax.ShapeDtypeStruct((RT, D), f32),
    )(acc3, g3, dinv, W3, b3)

    pooled = pl.pallas_call(
        _att_body,
        grid=(2,),
        in_specs=[
            pl.BlockSpec((1, N, D), lambda g: (g, 0, 0)),
            pl.BlockSpec((D, D), lambda g: (0, 0)),
        ],
        out_specs=pl.BlockSpec((8, D), lambda g: (g, 0)),
        out_shape=jax.ShapeDtypeStruct((16, D), f32),
    )(y3.reshape(2, N, D), W_att)
    pooled = pooled[::8, :]

    hi = pooled[0:1, :]
    hj = pooled[1, :]
    hj_rep = jnp.tile(hj, 16)[None, :]
    hcat = jnp.concatenate([pooled[0], hj])[None, :]
    wt = jnp.transpose(W_ntn, (1, 0, 2)).reshape(D, 16 * D)
    sel = jnp.repeat(jnp.eye(16, dtype=f32), D, axis=0)
    vt = V_ntn.T

    score = pl.pallas_call(
        _head_body, out_shape=jax.ShapeDtypeStruct((1, 1), f32),
    )(hi, hj_rep, hcat, wt, sel, vt, b_ntn[None, :],
      M1w, M1b[None, :], M2w, M2b[None, :], M3w, M3b[None, :],
      M4w, M4b[None, :], score_w, score_b[None, :])
    return score[0]
